# baseline (speedup 1.0000x reference)
import functools

import jax
import jax.numpy as jnp
from jax import lax
from jax.experimental import pallas as pl
from jax.experimental.pallas import tpu as pltpu
from jax.experimental.pallas import tpu_sc as plsc

_NC, _NS = 2, 16
_NW = _NC * _NS
_GCHUNK = 128
_RING_ROWS = 640
_NBUF = 2


def _sc_gather(xf, word_table):
    N = xf.shape[0]
    V, D = word_table.shape
    n_per_w = N // _NW
    n_chunks = n_per_w // _RING_ROWS
    assert n_per_w * _NW == N and n_chunks * _RING_ROWS == n_per_w
    assert n_chunks % _NBUF == 0
    n_gath = _RING_ROWS // _GCHUNK

    mesh = plsc.VectorSubcoreMesh(core_axis_name="c", subcore_axis_name="s")

    @functools.partial(
        pl.kernel,
        out_type=jax.ShapeDtypeStruct((N, D), jnp.float32),
        mesh=mesh,
        scratch_types=[
            pltpu.VMEM((n_per_w,), jnp.int32),
            pltpu.VMEM((_NBUF, _RING_ROWS, D), jnp.float32),
            pltpu.SemaphoreType.DMA,
            pltpu.SemaphoreType.DMA,
            pltpu.SemaphoreType.DMA,
            pltpu.SemaphoreType.DMA,
        ],
        compiler_params=pltpu.CompilerParams(use_tc_tiling_on_sc=False),
    )
    def gather_kernel(x_hbm, word_hbm, out_hbm, idx_v, rows_v,
                      semg0, semg1, semw0, semw1):
        wid = lax.axis_index("s") * _NC + lax.axis_index("c")
        base = wid * n_per_w
        semg = (semg0, semg1)
        semw = (semw0, semw1)

        pltpu.sync_copy(x_hbm.at[pl.ds(base, n_per_w)], idx_v)

        def fire(buf, step):
            off = step * _RING_ROWS
            for j in range(n_gath):
                pltpu.async_copy(
                    word_hbm.at[idx_v.at[pl.ds(off + j * _GCHUNK, _GCHUNK)]],
                    rows_v.at[buf, pl.ds(j * _GCHUNK, _GCHUNK)],
                    semg[buf],
                )

        def drain_g(buf):
            pltpu.make_async_copy(
                word_hbm.at[pl.ds(0, _RING_ROWS)], rows_v.at[buf], semg[buf]
            ).wait()

        def write_async(buf, step):
            pltpu.async_copy(
                rows_v.at[buf],
                out_hbm.at[pl.ds(base + step * _RING_ROWS, _RING_ROWS)],
                semw[buf],
            )

        def wait_w(buf):
            pltpu.make_async_copy(
                rows_v.at[buf], out_hbm.at[pl.ds(0, _RING_ROWS)], semw[buf]
            ).wait()

        fire(0, 0)

        def outer(i, carry):
            s0 = i * _NBUF

            @pl.when(i > 0)
            def _():
                wait_w(1)

            fire(1, s0 + 1)
            drain_g(0)
            write_async(0, s0)

            @pl.when(s0 + 2 < n_chunks)
            def _():
                wait_w(0)
                fire(0, s0 + 2)

            drain_g(1)
            write_async(1, s0 + 1)
            return carry

        lax.fori_loop(0, n_chunks // _NBUF, outer, 0)
        wait_w(0)
        wait_w(1)

    return gather_kernel(xf, word_table)


_SBLK = 2
_P = 4


def _make_ln_body(B, D):
    def _ln_body(g_ref, pos_ref, gamma_ref, beta_ref, *rest):
        o_ref = rest[-1]
        for k in range(_SBLK):
            h = g_ref[pl.ds(k * B, B), :].T + pos_ref[pl.ds(k * D, D), :]
            mean = jnp.mean(h, axis=0, keepdims=True)
            d = h - mean
            var = jnp.mean(d * d, axis=0, keepdims=True)
            o_ref[k] = (
                d * lax.rsqrt(var + jnp.float32(1e-5)) * gamma_ref[...]
                + beta_ref[...]
            )

    return _ln_body


def _tc_layernorm_piece(g2, pos2, gammaT, betaT, prev, piece, S, B, D):
    Sp = g2.shape[0] // B
    blk_off = piece * (Sp // _SBLK)
    in_specs = [
        pl.BlockSpec((_SBLK * B, D), lambda s: (s, 0)),
        pl.BlockSpec((_SBLK * D, 1), lambda s: (s, 0)),
        pl.BlockSpec((D, 1), lambda s: (0, 0)),
        pl.BlockSpec((D, 1), lambda s: (0, 0)),
    ]
    args = [g2, pos2, gammaT, betaT]
    kwargs = {}
    if prev is not None:
        in_specs.append(pl.BlockSpec((_SBLK, D, B), lambda s: (0, 0, 0)))
        args.append(prev)
        kwargs["input_output_aliases"] = {4: 0}
    return pl.pallas_call(
        _make_ln_body(B, D),
        grid=(Sp // _SBLK,),
        in_specs=in_specs,
        out_specs=pl.BlockSpec((_SBLK, D, B), lambda s: (s + blk_off, 0, 0)),
        out_shape=jax.ShapeDtypeStruct((S, D, B), jnp.float32),
        **kwargs,
    )(*args)


def kernel(x, word_table, pos_table, gamma, beta):
    B, S = x.shape
    V, D = word_table.shape
    N = B * S
    xT = jnp.swapaxes(x, 0, 1).reshape(N)
    pos2 = pos_table.reshape(S * D, 1)
    gammaT = gamma.reshape(D, 1)
    betaT = beta.reshape(D, 1)
    Sp = S // _P
    out_phys = None
    for p in range(_P):
        xp = xT[p * Sp * B:(p + 1) * Sp * B]
        g = _sc_gather(xp, word_table)
        out_phys = _tc_layernorm_piece(
            g,
            pos2[p * Sp * D:(p + 1) * Sp * D],
            gammaT,
            betaT,
            out_phys,
            p,
            S, B, D,
        )
    return jnp.transpose(out_phys, (2, 0, 1))


# speedup vs baseline: 1.0025x; 1.0025x over previous
import functools

import jax
import jax.numpy as jnp
from jax import lax
from jax.experimental import pallas as pl
from jax.experimental.pallas import tpu as pltpu
from jax.experimental.pallas import tpu_sc as plsc

_NC, _NS = 2, 16
_NW = _NC * _NS
_GCHUNK = 128
_RING_ROWS = 640
_NBUF = 2


def _sc_gather(xf, word_table):
    N = xf.shape[0]
    V, D = word_table.shape
    n_per_w = N // _NW
    n_chunks = n_per_w // _RING_ROWS
    assert n_per_w * _NW == N and n_chunks * _RING_ROWS == n_per_w
    assert n_chunks % _NBUF == 0
    n_gath = _RING_ROWS // _GCHUNK

    mesh = plsc.VectorSubcoreMesh(core_axis_name="c", subcore_axis_name="s")

    @functools.partial(
        pl.kernel,
        out_type=jax.ShapeDtypeStruct((N, D), jnp.float32),
        mesh=mesh,
        scratch_types=[
            pltpu.VMEM((n_per_w,), jnp.int32),
            pltpu.VMEM((_NBUF, _RING_ROWS, D), jnp.float32),
            pltpu.SemaphoreType.DMA,
            pltpu.SemaphoreType.DMA,
            pltpu.SemaphoreType.DMA,
            pltpu.SemaphoreType.DMA,
        ],
        compiler_params=pltpu.CompilerParams(use_tc_tiling_on_sc=False),
    )
    def gather_kernel(x_hbm, word_hbm, out_hbm, idx_v, rows_v,
                      semg0, semg1, semw0, semw1):
        wid = lax.axis_index("s") * _NC + lax.axis_index("c")
        base = wid * n_per_w
        semg = (semg0, semg1)
        semw = (semw0, semw1)

        pltpu.sync_copy(x_hbm.at[pl.ds(base, n_per_w)], idx_v)

        def fire(buf, step):
            off = step * _RING_ROWS
            for j in range(n_gath):
                pltpu.async_copy(
                    word_hbm.at[idx_v.at[pl.ds(off + j * _GCHUNK, _GCHUNK)]],
                    rows_v.at[buf, pl.ds(j * _GCHUNK, _GCHUNK)],
                    semg[buf],
                )

        def drain_g(buf):
            pltpu.make_async_copy(
                word_hbm.at[pl.ds(0, _RING_ROWS)], rows_v.at[buf], semg[buf]
            ).wait()

        def write_async(buf, step):
            pltpu.async_copy(
                rows_v.at[buf],
                out_hbm.at[pl.ds(base + step * _RING_ROWS, _RING_ROWS)],
                semw[buf],
            )

        def wait_w(buf):
            pltpu.make_async_copy(
                rows_v.at[buf], out_hbm.at[pl.ds(0, _RING_ROWS)], semw[buf]
            ).wait()

        fire(0, 0)

        def outer(i, carry):
            s0 = i * _NBUF

            @pl.when(i > 0)
            def _():
                wait_w(1)

            fire(1, s0 + 1)
            drain_g(0)
            write_async(0, s0)

            @pl.when(s0 + 2 < n_chunks)
            def _():
                wait_w(0)
                fire(0, s0 + 2)

            drain_g(1)
            write_async(1, s0 + 1)
            return carry

        lax.fori_loop(0, n_chunks // _NBUF, outer, 0)
        wait_w(0)
        wait_w(1)

    return gather_kernel(xf, word_table)


_SBLK = 2
_P = 5


def _make_ln_body(B, D):
    def _ln_body(g_ref, pos_ref, gamma_ref, beta_ref, *rest):
        o_ref = rest[-1]
        for k in range(_SBLK):
            h = g_ref[pl.ds(k * B, B), :].T + pos_ref[pl.ds(k * D, D), :]
            mean = jnp.mean(h, axis=0, keepdims=True)
            d = h - mean
            var = jnp.mean(d * d, axis=0, keepdims=True)
            o_ref[k] = (
                d * lax.rsqrt(var + jnp.float32(1e-5)) * gamma_ref[...]
                + beta_ref[...]
            )

    return _ln_body


def _tc_layernorm_piece(g2, pos2, gammaT, betaT, prev, piece, S, B, D):
    Sp = g2.shape[0] // B
    blk_off = piece * (Sp // _SBLK)
    in_specs = [
        pl.BlockSpec((_SBLK * B, D), lambda s: (s, 0)),
        pl.BlockSpec((_SBLK * D, 1), lambda s: (s, 0)),
        pl.BlockSpec((D, 1), lambda s: (0, 0)),
        pl.BlockSpec((D, 1), lambda s: (0, 0)),
    ]
    args = [g2, pos2, gammaT, betaT]
    kwargs = {}
    if prev is not None:
        in_specs.append(pl.BlockSpec((_SBLK, D, B), lambda s: (0, 0, 0)))
        args.append(prev)
        kwargs["input_output_aliases"] = {4: 0}
    return pl.pallas_call(
        _make_ln_body(B, D),
        grid=(Sp // _SBLK,),
        in_specs=in_specs,
        out_specs=pl.BlockSpec((_SBLK, D, B), lambda s: (s + blk_off, 0, 0)),
        out_shape=jax.ShapeDtypeStruct((S, D, B), jnp.float32),
        **kwargs,
    )(*args)


def kernel(x, word_table, pos_table, gamma, beta):
    B, S = x.shape
    V, D = word_table.shape
    N = B * S
    xT = jnp.swapaxes(x, 0, 1).reshape(N)
    pos2 = pos_table.reshape(S * D, 1)
    gammaT = gamma.reshape(D, 1)
    betaT = beta.reshape(D, 1)
    Sp = S // _P
    out_phys = None
    for p in range(_P):
        xp = xT[p * Sp * B:(p + 1) * Sp * B]
        g = _sc_gather(xp, word_table)
        out_phys = _tc_layernorm_piece(
            g,
            pos2[p * Sp * D:(p + 1) * Sp * D],
            gammaT,
            betaT,
            out_phys,
            p,
            S, B, D,
        )
    return jnp.transpose(out_phys, (2, 0, 1))
